# Initial kernel scaffold; baseline (speedup 1.0000x reference)
import jax
import jax.numpy as jnp
from jax.experimental import pallas as pl


def kernel(center_ids, context_ids, in_embed, out_embed):
    raise NotImplementedError("write your pallas kernel here")


# speedup vs baseline: 1.7497x; 1.7497x over previous
import functools

import jax
import jax.numpy as jnp
from jax import lax
from jax.experimental import pallas as pl
from jax.experimental.pallas import tpu as pltpu
from jax.experimental.pallas import tpu_sc as plsc

_VOCAB = 1000000
_D = 32
_B = 16384
_L = 50

_NC = 2
_NS = 16
_NW = _NC * _NS
_BPW = _B // _NW
_ITEMS = _BPW * _L
_IDXROW = 2 * _L
_IDXROWS_W = _ITEMS // _IDXROW
_GB = 16
_GROUP_ITEMS = _GB * _L
_GROUP_DMAS = _GROUP_ITEMS // _IDXROW
_NG = _BPW // _GB
_CEN_COLS = 128


def _sc_scores_body(cen_hbm, ctx_hbm, in_hbm, out_hbm, scores_hbm,
                    cidx_v, ctr_v, cxidx_v, ctx_a, ctx_b, sbuf_v,
                    sem_c, sem_a, sem_b):
    wid = lax.axis_index("s") * _NC + lax.axis_index("c")
    iota = lax.iota(jnp.int32, 16)

    pltpu.sync_copy(cen_hbm.at[pl.ds(wid * 4, 4)], cidx_v)
    ctr_copies = [
        pltpu.make_async_copy(in_hbm.at[cidx_v.at[j]],
                              ctr_v.at[pl.ds(j * _CEN_COLS, _CEN_COLS)],
                              sem_c)
        for j in range(4)
    ]
    for c in ctr_copies:
        c.start()
    pltpu.sync_copy(ctx_hbm.at[pl.ds(wid * _IDXROWS_W, _IDXROWS_W)], cxidx_v)
    for c in ctr_copies:
        c.wait()

    def fire_ctx(g, ctx_buf, sem):
        for j in range(_GROUP_DMAS):
            pltpu.make_async_copy(
                out_hbm.at[cxidx_v.at[g * _GROUP_DMAS + j]],
                ctx_buf.at[pl.ds(j * _IDXROW, _IDXROW)],
                sem).start()

    def drain_ctx(g, ctx_buf, sem):
        for j in range(_GROUP_DMAS):
            pltpu.make_async_copy(
                out_hbm.at[cxidx_v.at[g * _GROUP_DMAS + j]],
                ctx_buf.at[pl.ds(j * _IDXROW, _IDXROW)],
                sem).wait()

    def compute_group(g, ctx_buf):
        b0 = g * _GB
        cvec = [
            plsc.load_gather(ctr_v, [b0 + iota, jnp.full((16,), d, jnp.int32)])
            for d in range(_D)
        ]
        sbase = g * _GROUP_ITEMS + iota * _L

        def l_body(l, carry):
            ridx = iota * _L + l
            acc = cvec[0] * plsc.load_gather(
                ctx_buf, [ridx, jnp.zeros((16,), jnp.int32)])
            for d in range(1, _D):
                acc = acc + cvec[d] * plsc.load_gather(
                    ctx_buf, [ridx, jnp.full((16,), d, jnp.int32)])
            plsc.store_scatter(sbuf_v, [sbase + l], acc)
            return carry

        lax.fori_loop(0, _L, l_body, 0)

    fire_ctx(0, ctx_a, sem_a)

    def outer(k, carry):
        g0 = 2 * k
        fire_ctx(g0 + 1, ctx_b, sem_b)
        drain_ctx(g0, ctx_a, sem_a)
        compute_group(g0, ctx_a)

        @pl.when(k < _NG // 2 - 1)
        def _():
            fire_ctx(g0 + 2, ctx_a, sem_a)

        drain_ctx(g0 + 1, ctx_b, sem_b)
        compute_group(g0 + 1, ctx_b)
        return carry

    lax.fori_loop(0, _NG // 2, outer, 0)
    pltpu.sync_copy(sbuf_v, scores_hbm.at[pl.ds(wid * _ITEMS, _ITEMS)])


def _sc_scores(cen2d, ctx2d, in_embed, out_embed):
    mesh = plsc.VectorSubcoreMesh(core_axis_name="c", subcore_axis_name="s")
    fn = pl.kernel(
        _sc_scores_body,
        out_type=jax.ShapeDtypeStruct((_B * _L,), jnp.float32),
        mesh=mesh,
        scratch_types=[
            pltpu.VMEM((4, _CEN_COLS), jnp.int32),
            pltpu.VMEM((_BPW, _D), jnp.float32),
            pltpu.VMEM((_IDXROWS_W, _IDXROW), jnp.int32),
            pltpu.VMEM((_GROUP_ITEMS, _D), jnp.float32),
            pltpu.VMEM((_GROUP_ITEMS, _D), jnp.float32),
            pltpu.VMEM((_ITEMS,), jnp.float32),
            pltpu.SemaphoreType.DMA,
            pltpu.SemaphoreType.DMA,
            pltpu.SemaphoreType.DMA,
        ],
        compiler_params=pltpu.CompilerParams(
            needs_layout_passes=False, use_tc_tiling_on_sc=False),
    )
    return fn(cen2d, ctx2d, in_embed, out_embed)


def _tc_loss_body(x_ref, o_ref):
    t = -x_ref[...]
    sp = jnp.maximum(t, 0.0) + jnp.log(1.0 + jnp.exp(-jnp.abs(t)))
    o_ref[0, 0] = jnp.sum(sp) * (1.0 / (_B * _L))


def _tc_loss(scores2d):
    return pl.pallas_call(
        _tc_loss_body,
        out_shape=jax.ShapeDtypeStruct((1, 1), jnp.float32),
        out_specs=pl.BlockSpec(memory_space=pltpu.SMEM),
    )(scores2d)


def kernel(center_ids, context_ids, in_embed, out_embed):
    cen2d = center_ids.astype(jnp.int32).reshape(_B // _CEN_COLS, _CEN_COLS)
    ctx2d = context_ids.astype(jnp.int32).reshape(_B * _L // _IDXROW, _IDXROW)
    scores = _sc_scores(cen2d, ctx2d, in_embed, out_embed)
    loss2d = _tc_loss(scores.reshape(_B * _L // 128, 128))
    return loss2d[0, 0]
